# Initial kernel scaffold; baseline (speedup 1.0000x reference)
import jax
import jax.numpy as jnp
from jax.experimental import pallas as pl


def kernel(x, edge_index, W1, b1, g1, be1, W2, b2, g2, be2, Wc, bc):
    raise NotImplementedError("write your pallas kernel here")


# speedup vs baseline: 12.1415x; 12.1415x over previous
import functools

import jax
import jax.numpy as jnp
from jax import lax
from jax.experimental import pallas as pl
from jax.experimental.pallas import tpu as pltpu
from jax.experimental.pallas import tpu_sc as plsc

N = 10000
E = 320000
D = 128
H = 128
C = 40
EPS = 1e-5

NC = 2
NS = 16
NW = NC * NS
EPT = E // NW
CH = 80
NCHUNK = EPT // CH
NP = 10240
RPT = NP // NS

_mesh = plsc.VectorSubcoreMesh(
    core_axis_name="c", subcore_axis_name="s", num_cores=NC, num_subcores=NS)


@functools.partial(
    pl.kernel,
    out_type=jax.ShapeDtypeStruct((NC, NP, H), jnp.float32),
    mesh=_mesh,
    scratch_types=[
        pltpu.VMEM((CH,), jnp.int32),
        pltpu.VMEM((CH, H), jnp.float32),
        pltpu.VMEM_SHARED((NP, H), jnp.float32),
    ],
)
def _deg_kernel(dst_hbm, zeros_hbm, ones_hbm, out_hbm, dstc_v, ones_v, acc_sh):
    cid = lax.axis_index("c")
    sid = lax.axis_index("s")
    wid = sid * NC + cid
    base = wid * EPT

    pltpu.sync_copy(ones_hbm, ones_v)
    pltpu.sync_copy(zeros_hbm.at[pl.ds(sid * RPT, RPT)],
                    acc_sh.at[pl.ds(sid * RPT, RPT)])
    plsc.subcore_barrier()

    @pl.loop(0, NCHUNK)
    def _chunks(j):
        off = pl.multiple_of(base + j * CH, 8)
        pltpu.sync_copy(dst_hbm.at[pl.ds(off, CH)], dstc_v)
        pltpu.sync_copy(ones_v, acc_sh.at[dstc_v], add=True)

    plsc.subcore_barrier()
    pltpu.sync_copy(acc_sh.at[pl.ds(sid * RPT, RPT)],
                    out_hbm.at[cid, pl.ds(sid * RPT, RPT)])


@functools.partial(
    pl.kernel,
    out_type=jax.ShapeDtypeStruct((NC, NP, H), jnp.float32),
    mesh=_mesh,
    scratch_types=[
        pltpu.VMEM((CH,), jnp.int32),
        pltpu.VMEM((CH,), jnp.int32),
        pltpu.VMEM((CH, H), jnp.float32),
        pltpu.VMEM_SHARED((NP, H), jnp.float32),
        pltpu.SemaphoreType.DMA,
    ],
)
def _agg_kernel(table_hbm, src_hbm, dst_hbm, zeros_hbm, out_hbm,
                srcc_v, dstc_v, rows_v, acc_sh, sem):
    cid = lax.axis_index("c")
    sid = lax.axis_index("s")
    wid = sid * NC + cid
    base = wid * EPT

    pltpu.sync_copy(zeros_hbm.at[pl.ds(sid * RPT, RPT)],
                    acc_sh.at[pl.ds(sid * RPT, RPT)])
    plsc.subcore_barrier()

    @pl.loop(0, NCHUNK)
    def _chunks(j):
        off = pl.multiple_of(base + j * CH, 8)
        pltpu.sync_copy(src_hbm.at[pl.ds(off, CH)], srcc_v)
        pltpu.async_copy(table_hbm.at[srcc_v], rows_v, sem).wait()
        pltpu.sync_copy(dst_hbm.at[pl.ds(off, CH)], dstc_v)
        pltpu.sync_copy(rows_v, acc_sh.at[dstc_v], add=True)

    plsc.subcore_barrier()
    pltpu.sync_copy(acc_sh.at[pl.ds(sid * RPT, RPT)],
                    out_hbm.at[cid, pl.ds(sid * RPT, RPT)])


def _dis_from_degp(degp):
    deg = 1.0 + degp[0, :N, 0:1] + degp[1, :N, 0:1]
    return lax.rsqrt(deg)


def _tc1_body(x_ref, w1_ref, degp_ref, o_ref):
    dis = _dis_from_degp(degp_ref[...])
    h = jnp.dot(x_ref[...], w1_ref[...], preferred_element_type=jnp.float32)
    o_ref[...] = h * dis


_tc1 = pl.pallas_call(
    _tc1_body, out_shape=jax.ShapeDtypeStruct((N, H), jnp.float32))


def _bn_relu(z, g, be):
    mean = jnp.mean(z, axis=0, keepdims=True)
    c = z - mean
    var = jnp.mean(c * c, axis=0, keepdims=True)
    return jnp.maximum(c * lax.rsqrt(var + EPS) * g + be, 0.0)


def _tc2_body(part_ref, hp_ref, degp_ref, b_ref, g_ref, be_ref, w2_ref, o_ref):
    dis = _dis_from_degp(degp_ref[...])
    part = part_ref[...]
    z = (part[0, :N] + part[1, :N] + hp_ref[...]) * dis + b_ref[...]
    y = _bn_relu(z, g_ref[...], be_ref[...])
    o_ref[...] = jnp.dot(y, w2_ref[...], preferred_element_type=jnp.float32) * dis


_tc2 = pl.pallas_call(
    _tc2_body, out_shape=jax.ShapeDtypeStruct((N, H), jnp.float32))


def _tc3_body(part_ref, hp_ref, degp_ref, b_ref, g_ref, be_ref, wc_ref,
              bc_ref, o_ref):
    dis = _dis_from_degp(degp_ref[...])
    part = part_ref[...]
    z = (part[0, :N] + part[1, :N] + hp_ref[...]) * dis + b_ref[...]
    y = _bn_relu(z, g_ref[...], be_ref[...])
    logits = jnp.dot(y, wc_ref[...], preferred_element_type=jnp.float32) + bc_ref[...]
    m = jnp.max(logits, axis=1, keepdims=True)
    s = logits - m
    lse = jnp.log(jnp.sum(jnp.exp(s), axis=1, keepdims=True))
    o_ref[...] = s - lse


_tc3 = pl.pallas_call(
    _tc3_body, out_shape=jax.ShapeDtypeStruct((N, C), jnp.float32))


def kernel(x, edge_index, W1, b1, g1, be1, W2, b2, g2, be2, Wc, bc):
    ei = edge_index.astype(jnp.int32)
    src = ei[0]
    dst = ei[1]
    zeros_nh = jnp.zeros((NP, H), jnp.float32)
    ones_ch = jnp.ones((CH, H), jnp.float32)

    degp = _deg_kernel(dst, zeros_nh, ones_ch)[:, :, :16]
    h1p = _tc1(x, W1, degp)
    pA = _agg_kernel(h1p, src, dst, zeros_nh)
    h2p = _tc2(pA, h1p, degp, b1.reshape(1, H), g1.reshape(1, H),
               be1.reshape(1, H), W2)
    pB = _agg_kernel(h2p, src, dst, zeros_nh)
    out = _tc3(pB, h2p, degp, b2.reshape(1, H), g2.reshape(1, H),
               be2.reshape(1, H), Wc, bc.reshape(1, C))
    return out


# speedup vs baseline: 25.3965x; 2.0917x over previous
import functools

import jax
import jax.numpy as jnp
from jax import lax
from jax.experimental import pallas as pl
from jax.experimental.pallas import tpu as pltpu
from jax.experimental.pallas import tpu_sc as plsc

N = 10000
E = 320000
D = 128
H = 128
C = 40
EPS = 1e-5

NC = 2
NS = 16
NW = NC * NS
EPT = E // NW
CH = 80
NCHUNK = EPT // CH
NBUF = 2
NP = 10240
RPT = NP // NS

_mesh = plsc.VectorSubcoreMesh(
    core_axis_name="c", subcore_axis_name="s", num_cores=NC, num_subcores=NS)


@functools.partial(
    pl.kernel,
    out_type=jax.ShapeDtypeStruct((NC, NP, H), jnp.float32),
    mesh=_mesh,
    scratch_types=[
        pltpu.VMEM((NCHUNK, CH), jnp.int32),
        pltpu.VMEM((CH, H), jnp.float32),
        pltpu.VMEM_SHARED((NP, H), jnp.float32),
    ],
)
def _deg_kernel(dst3_hbm, zeros_hbm, ones_hbm, out_hbm, dsts_v, ones_v, acc_sh):
    cid = lax.axis_index("c")
    sid = lax.axis_index("s")
    wid = sid * NC + cid

    pltpu.sync_copy(ones_hbm, ones_v)
    pltpu.sync_copy(dst3_hbm.at[wid], dsts_v)
    pltpu.sync_copy(zeros_hbm.at[pl.ds(sid * RPT, RPT)],
                    acc_sh.at[pl.ds(sid * RPT, RPT)])
    plsc.subcore_barrier()

    @pl.loop(0, NCHUNK)
    def _chunks(j):
        pltpu.sync_copy(ones_v, acc_sh.at[dsts_v.at[j]], add=True)

    plsc.subcore_barrier()
    pltpu.sync_copy(acc_sh.at[pl.ds(sid * RPT, RPT)],
                    out_hbm.at[cid, pl.ds(sid * RPT, RPT)])


@functools.partial(
    pl.kernel,
    out_type=jax.ShapeDtypeStruct((NC, NP, H), jnp.float32),
    mesh=_mesh,
    scratch_types=[
        pltpu.VMEM((EPT,), jnp.int32),
        pltpu.VMEM((NCHUNK, CH), jnp.int32),
        pltpu.VMEM((NBUF, CH, H), jnp.float32),
        pltpu.VMEM_SHARED((NP, H), jnp.float32),
        pltpu.SemaphoreType.DMA,
        pltpu.SemaphoreType.DMA,
    ],
)
def _agg_kernel(table_hbm, src1_hbm, dst3_hbm, zeros_hbm, out_hbm,
                srcs_v, dsts_v, rows_v, acc_sh, sem0, sem1):
    sems = [sem0, sem1]
    cid = lax.axis_index("c")
    sid = lax.axis_index("s")
    wid = sid * NC + cid

    pltpu.sync_copy(src1_hbm.at[pl.ds(wid * EPT, EPT)], srcs_v)
    pltpu.sync_copy(dst3_hbm.at[wid], dsts_v)
    pltpu.sync_copy(zeros_hbm.at[pl.ds(sid * RPT, RPT)],
                    acc_sh.at[pl.ds(sid * RPT, RPT)])
    plsc.subcore_barrier()

    for b in range(NBUF):
        pltpu.async_copy(table_hbm.at[srcs_v.at[pl.ds(b * CH, CH)]],
                         rows_v.at[b], sems[b])

    @pl.loop(0, NCHUNK, step=NBUF)
    def _outer(o):
        for b in range(NBUF):
            j = o + b

            @pl.when(j < NCHUNK)
            def _():
                pltpu.make_async_copy(
                    table_hbm.at[srcs_v.at[pl.ds(j * CH, CH)]],
                    rows_v.at[b], sems[b]).wait()
                pltpu.sync_copy(rows_v.at[b], acc_sh.at[dsts_v.at[j]],
                                add=True)

                @pl.when(j + NBUF < NCHUNK)
                def _():
                    pltpu.async_copy(
                        table_hbm.at[srcs_v.at[pl.ds((j + NBUF) * CH, CH)]],
                        rows_v.at[b], sems[b])

    plsc.subcore_barrier()
    pltpu.sync_copy(acc_sh.at[pl.ds(sid * RPT, RPT)],
                    out_hbm.at[cid, pl.ds(sid * RPT, RPT)])


def _dis_from_degp(degp):
    deg = 1.0 + degp[0, :N, 0:1] + degp[1, :N, 0:1]
    return lax.rsqrt(deg)


def _tc1_body(x_ref, w1_ref, degp_ref, o_ref):
    dis = _dis_from_degp(degp_ref[...])
    h = jnp.dot(x_ref[...], w1_ref[...], preferred_element_type=jnp.float32)
    o_ref[...] = h * dis


_tc1 = pl.pallas_call(
    _tc1_body, out_shape=jax.ShapeDtypeStruct((N, H), jnp.float32))


def _bn_relu(z, g, be):
    mean = jnp.mean(z, axis=0, keepdims=True)
    c = z - mean
    var = jnp.mean(c * c, axis=0, keepdims=True)
    return jnp.maximum(c * lax.rsqrt(var + EPS) * g + be, 0.0)


def _tc2_body(part_ref, hp_ref, degp_ref, b_ref, g_ref, be_ref, w2_ref, o_ref):
    dis = _dis_from_degp(degp_ref[...])
    part = part_ref[...]
    z = (part[0, :N] + part[1, :N] + hp_ref[...]) * dis + b_ref[...]
    y = _bn_relu(z, g_ref[...], be_ref[...])
    o_ref[...] = jnp.dot(y, w2_ref[...], preferred_element_type=jnp.float32) * dis


_tc2 = pl.pallas_call(
    _tc2_body, out_shape=jax.ShapeDtypeStruct((N, H), jnp.float32))


def _tc3_body(part_ref, hp_ref, degp_ref, b_ref, g_ref, be_ref, wc_ref,
              bc_ref, o_ref):
    dis = _dis_from_degp(degp_ref[...])
    part = part_ref[...]
    z = (part[0, :N] + part[1, :N] + hp_ref[...]) * dis + b_ref[...]
    y = _bn_relu(z, g_ref[...], be_ref[...])
    logits = jnp.dot(y, wc_ref[...], preferred_element_type=jnp.float32) + bc_ref[...]
    m = jnp.max(logits, axis=1, keepdims=True)
    s = logits - m
    lse = jnp.log(jnp.sum(jnp.exp(s), axis=1, keepdims=True))
    o_ref[...] = s - lse


_tc3 = pl.pallas_call(
    _tc3_body, out_shape=jax.ShapeDtypeStruct((N, C), jnp.float32))


def kernel(x, edge_index, W1, b1, g1, be1, W2, b2, g2, be2, Wc, bc):
    ei = edge_index.astype(jnp.int32)
    src1 = ei[0]
    dst2 = ei[1].reshape(NW, NCHUNK, CH)
    zeros_nh = jnp.zeros((NP, H), jnp.float32)
    ones_ch = jnp.ones((CH, H), jnp.float32)

    degp = _deg_kernel(dst2, zeros_nh, ones_ch)[:, :, :16]
    h1p = _tc1(x, W1, degp)
    pA = _agg_kernel(h1p, src1, dst2, zeros_nh)
    h2p = _tc2(pA, h1p, degp, b1.reshape(1, H), g1.reshape(1, H),
               be1.reshape(1, H), W2)
    pB = _agg_kernel(h2p, src1, dst2, zeros_nh)
    out = _tc3(pB, h2p, degp, b2.reshape(1, H), g2.reshape(1, H),
               be2.reshape(1, H), Wc, bc.reshape(1, C))
    return out


# speedup vs baseline: 28.9624x; 1.1404x over previous
import functools

import jax
import jax.numpy as jnp
from jax import lax
from jax.experimental import pallas as pl
from jax.experimental.pallas import tpu as pltpu
from jax.experimental.pallas import tpu_sc as plsc

N = 10000
E = 320000
D = 128
H = 128
C = 40
EPS = 1e-5

NC = 2
NS = 16
NW = NC * NS
EPT = E // NW
CH = 80
NCHUNK = EPT // CH
NBUF = 2
NP = 10240
RPT = NP // NS

_mesh = plsc.VectorSubcoreMesh(
    core_axis_name="c", subcore_axis_name="s", num_cores=NC, num_subcores=NS)


@functools.partial(
    pl.kernel,
    out_type=jax.ShapeDtypeStruct((NC, 1, NP), jnp.float32),
    mesh=_mesh,
    scratch_types=[
        pltpu.VMEM((NCHUNK, CH), jnp.int32),
        pltpu.VMEM((CH,), jnp.float32),
        pltpu.VMEM_SHARED((NP,), jnp.float32),
    ],
)
def _deg_kernel(dst3_hbm, zeros_hbm, ones_hbm, out_hbm, dsts_v, ones_v, acc_sh):
    cid = lax.axis_index("c")
    sid = lax.axis_index("s")
    wid = sid * NC + cid

    pltpu.sync_copy(ones_hbm, ones_v)
    pltpu.sync_copy(dst3_hbm.at[wid], dsts_v)
    pltpu.sync_copy(zeros_hbm.at[pl.ds(sid * RPT, RPT)],
                    acc_sh.at[pl.ds(sid * RPT, RPT)])
    plsc.subcore_barrier()

    @pl.loop(0, NCHUNK)
    def _chunks(j):
        pltpu.sync_copy(ones_v, acc_sh.at[dsts_v.at[j]], add=True)

    plsc.subcore_barrier()
    pltpu.sync_copy(acc_sh.at[pl.ds(sid * RPT, RPT)],
                    out_hbm.at[cid, 0, pl.ds(sid * RPT, RPT)])


@functools.partial(
    pl.kernel,
    out_type=jax.ShapeDtypeStruct((NC, NP, H), jnp.float32),
    mesh=_mesh,
    scratch_types=[
        pltpu.VMEM((EPT,), jnp.int32),
        pltpu.VMEM((NCHUNK, CH), jnp.int32),
        pltpu.VMEM((NBUF, CH, H), jnp.float32),
        pltpu.VMEM_SHARED((NP, H), jnp.float32),
        pltpu.SemaphoreType.DMA,
        pltpu.SemaphoreType.DMA,
    ],
)
def _agg_kernel(table_hbm, src1_hbm, dst3_hbm, zeros_hbm, out_hbm,
                srcs_v, dsts_v, rows_v, acc_sh, sem0, sem1):
    sems = [sem0, sem1]
    cid = lax.axis_index("c")
    sid = lax.axis_index("s")
    wid = sid * NC + cid

    pltpu.sync_copy(src1_hbm.at[pl.ds(wid * EPT, EPT)], srcs_v)
    pltpu.sync_copy(dst3_hbm.at[wid], dsts_v)
    pltpu.sync_copy(zeros_hbm.at[pl.ds(sid * RPT, RPT)],
                    acc_sh.at[pl.ds(sid * RPT, RPT)])
    plsc.subcore_barrier()

    for b in range(NBUF):
        pltpu.async_copy(table_hbm.at[srcs_v.at[pl.ds(b * CH, CH)]],
                         rows_v.at[b], sems[b])

    @pl.loop(0, NCHUNK, step=NBUF)
    def _outer(o):
        for b in range(NBUF):
            j = o + b

            @pl.when(j < NCHUNK)
            def _():
                pltpu.make_async_copy(
                    table_hbm.at[srcs_v.at[pl.ds(j * CH, CH)]],
                    rows_v.at[b], sems[b]).wait()
                pltpu.sync_copy(rows_v.at[b], acc_sh.at[dsts_v.at[j]],
                                add=True)

                @pl.when(j + NBUF < NCHUNK)
                def _():
                    pltpu.async_copy(
                        table_hbm.at[srcs_v.at[pl.ds((j + NBUF) * CH, CH)]],
                        rows_v.at[b], sems[b])

    plsc.subcore_barrier()
    pltpu.sync_copy(acc_sh.at[pl.ds(sid * RPT, RPT)],
                    out_hbm.at[cid, pl.ds(sid * RPT, RPT)])


def _dis_from_degp(degp):
    deg = 1.0 + degp[0] + degp[1]
    return lax.rsqrt(deg)


def _tc1_body(x_ref, w1_ref, degp_ref, o_ref):
    dis = _dis_from_degp(degp_ref[...])
    h = jnp.dot(x_ref[...], w1_ref[...], preferred_element_type=jnp.float32)
    o_ref[...] = h * dis


_tc1 = pl.pallas_call(
    _tc1_body, out_shape=jax.ShapeDtypeStruct((N, H), jnp.float32))


def _bn_relu(z, g, be):
    mean = jnp.mean(z, axis=0, keepdims=True)
    c = z - mean
    var = jnp.mean(c * c, axis=0, keepdims=True)
    return jnp.maximum(c * lax.rsqrt(var + EPS) * g + be, 0.0)


def _tc2_body(part_ref, hp_ref, degp_ref, b_ref, g_ref, be_ref, w2_ref, o_ref):
    dis = _dis_from_degp(degp_ref[...])
    part = part_ref[...]
    z = (part[0, :N] + part[1, :N] + hp_ref[...]) * dis + b_ref[...]
    y = _bn_relu(z, g_ref[...], be_ref[...])
    o_ref[...] = jnp.dot(y, w2_ref[...], preferred_element_type=jnp.float32) * dis


_tc2 = pl.pallas_call(
    _tc2_body, out_shape=jax.ShapeDtypeStruct((N, H), jnp.float32))


def _tc3_body(part_ref, hp_ref, degp_ref, b_ref, g_ref, be_ref, wc_ref,
              bc_ref, o_ref):
    dis = _dis_from_degp(degp_ref[...])
    part = part_ref[...]
    z = (part[0, :N] + part[1, :N] + hp_ref[...]) * dis + b_ref[...]
    y = _bn_relu(z, g_ref[...], be_ref[...])
    logits = jnp.dot(y, wc_ref[...], preferred_element_type=jnp.float32) + bc_ref[...]
    m = jnp.max(logits, axis=1, keepdims=True)
    s = logits - m
    lse = jnp.log(jnp.sum(jnp.exp(s), axis=1, keepdims=True))
    o_ref[...] = s - lse


_tc3 = pl.pallas_call(
    _tc3_body, out_shape=jax.ShapeDtypeStruct((N, C), jnp.float32))


def kernel(x, edge_index, W1, b1, g1, be1, W2, b2, g2, be2, Wc, bc):
    ei = edge_index.astype(jnp.int32)
    src1 = ei[0]
    dst1 = ei[1]
    dst2 = ei[1].reshape(NW, NCHUNK, CH)
    zeros_nh = jnp.zeros((NP, H), jnp.float32)
    zeros_1d = jnp.zeros((NP,), jnp.float32)
    ones_c = jnp.ones((CH,), jnp.float32)

    degp = _deg_kernel(dst2, zeros_1d, ones_c)
    degp = degp.reshape(NC, NP, 1)[:, :N]
    h1p = _tc1(x, W1, degp)
    pA = _agg_kernel(h1p, src1, dst2, zeros_nh)
    h2p = _tc2(pA, h1p, degp, b1.reshape(1, H), g1.reshape(1, H),
               be1.reshape(1, H), W2)
    pB = _agg_kernel(h2p, src1, dst2, zeros_nh)
    out = _tc3(pB, h2p, degp, b2.reshape(1, H), g2.reshape(1, H),
               be2.reshape(1, H), Wc, bc.reshape(1, C))
    return out


# speedup vs baseline: 32.3931x; 1.1185x over previous
import functools

import jax
import jax.numpy as jnp
from jax import lax
from jax.experimental import pallas as pl
from jax.experimental.pallas import tpu as pltpu
from jax.experimental.pallas import tpu_sc as plsc

N = 10000
E = 320000
D = 128
H = 128
C = 40
EPS = 1e-5

NC = 2
NS = 16
NW = NC * NS
EPT = E // NW
CH = 80
NCHUNK = EPT // CH
NBUF = 3
NG = (NCHUNK + 7) // 8
NCP = NG * 8
NP = 10240
RPT = NP // NS

_mesh = plsc.VectorSubcoreMesh(
    core_axis_name="c", subcore_axis_name="s", num_cores=NC, num_subcores=NS)


@functools.partial(
    pl.kernel,
    out_type=jax.ShapeDtypeStruct((NC, 1, NP), jnp.float32),
    mesh=_mesh,
    scratch_types=[
        pltpu.VMEM((NCP, CH), jnp.int32),
        pltpu.VMEM((CH,), jnp.float32),
        pltpu.VMEM_SHARED((NP,), jnp.float32),
    ],
)
def _deg_kernel(dst3_hbm, zeros_hbm, ones_hbm, out_hbm, dsts_v, ones_v, acc_sh):
    cid = lax.axis_index("c")
    sid = lax.axis_index("s")
    wid = sid * NC + cid

    pltpu.sync_copy(ones_hbm, ones_v)
    pltpu.sync_copy(dst3_hbm.at[wid], dsts_v)
    pltpu.sync_copy(zeros_hbm.at[pl.ds(sid * RPT, RPT)],
                    acc_sh.at[pl.ds(sid * RPT, RPT)])
    plsc.subcore_barrier()

    @pl.loop(0, NCHUNK)
    def _chunks(j):
        pltpu.sync_copy(ones_v, acc_sh.at[dsts_v.at[j]], add=True)

    plsc.subcore_barrier()
    pltpu.sync_copy(acc_sh.at[pl.ds(sid * RPT, RPT)],
                    out_hbm.at[cid, 0, pl.ds(sid * RPT, RPT)])


@functools.partial(
    pl.kernel,
    out_type=jax.ShapeDtypeStruct((NC, NP, H), jnp.float32),
    mesh=_mesh,
    scratch_types=[
        pltpu.VMEM((EPT,), jnp.int32),
        pltpu.VMEM((2, 8, CH), jnp.int32),
        pltpu.VMEM((NBUF, CH, H), jnp.float32),
        pltpu.VMEM_SHARED((NP, H), jnp.float32),
        pltpu.SemaphoreType.DMA,
        pltpu.SemaphoreType.DMA,
        pltpu.SemaphoreType.DMA,
        pltpu.SemaphoreType.DMA,
        pltpu.SemaphoreType.DMA,
        pltpu.SemaphoreType.DMA,
        pltpu.SemaphoreType.DMA,
        pltpu.SemaphoreType.DMA,
    ],
)
def _agg_kernel(table_hbm, src1_hbm, dst3_hbm, zeros_hbm, out_hbm,
                srcs_v, dring_v, rows_v, acc_sh,
                g0, g1, g2, s0, s1, s2, d0, d1):
    gsem = [g0, g1, g2]
    ssem = [s0, s1, s2]
    dsem = [d0, d1]
    cid = lax.axis_index("c")
    sid = lax.axis_index("s")
    wid = sid * NC + cid

    pltpu.sync_copy(src1_hbm.at[pl.ds(wid * EPT, EPT)], srcs_v)
    pltpu.sync_copy(dst3_hbm.at[wid, pl.ds(0, 8)], dring_v.at[0])
    pltpu.async_copy(dst3_hbm.at[wid, pl.ds(8, 8)], dring_v.at[1], dsem[1])
    pltpu.sync_copy(zeros_hbm.at[pl.ds(sid * RPT, RPT)],
                    acc_sh.at[pl.ds(sid * RPT, RPT)])
    plsc.subcore_barrier()

    for b in range(NBUF - 1):
        pltpu.async_copy(table_hbm.at[srcs_v.at[pl.ds(b * CH, CH)]],
                         rows_v.at[b], gsem[b])

    @pl.loop(0, NCHUNK, step=NBUF)
    def _outer(o):
        for b in range(NBUF):
            j = o + b

            @pl.when(j < NCHUNK)
            def _():
                g = lax.shift_right_logical(j, 3)
                gb = lax.bitwise_and(g, 1)
                r = lax.bitwise_and(j, 7)

                @pl.when((r == 0) & (g >= 1))
                def _():
                    @pl.when(gb == 0)
                    def _():
                        pltpu.make_async_copy(
                            dst3_hbm.at[wid, pl.ds(0, 8)], dring_v.at[0],
                            dsem[0]).wait()

                    @pl.when(gb == 1)
                    def _():
                        pltpu.make_async_copy(
                            dst3_hbm.at[wid, pl.ds(0, 8)], dring_v.at[1],
                            dsem[1]).wait()

                pltpu.make_async_copy(
                    table_hbm.at[srcs_v.at[pl.ds(j * CH, CH)]],
                    rows_v.at[b], gsem[b]).wait()
                pltpu.async_copy(rows_v.at[b], acc_sh.at[dring_v.at[gb, r]],
                                 ssem[b], add=True)

                k = j + NBUF - 1
                bk = (b + NBUF - 1) % NBUF

                @pl.when(k < NCHUNK)
                def _():
                    @pl.when(j >= 1)
                    def _():
                        pltpu.make_async_copy(
                            rows_v.at[bk], acc_sh.at[dring_v.at[0, 0]],
                            ssem[bk]).wait()

                    pltpu.async_copy(
                        table_hbm.at[srcs_v.at[pl.ds(k * CH, CH)]],
                        rows_v.at[bk], gsem[bk])

                @pl.when((r == 0) & (g >= 1) & (g < NG - 1))
                def _():
                    nxt = pl.multiple_of((g + 1) * 8, 8)

                    @pl.when(gb == 0)
                    def _():
                        pltpu.async_copy(dst3_hbm.at[wid, pl.ds(nxt, 8)],
                                         dring_v.at[1], dsem[1])

                    @pl.when(gb == 1)
                    def _():
                        pltpu.async_copy(dst3_hbm.at[wid, pl.ds(nxt, 8)],
                                         dring_v.at[0], dsem[0])

    for b in range(NBUF):
        pltpu.make_async_copy(rows_v.at[b], acc_sh.at[dring_v.at[0, 0]],
                              ssem[b]).wait()

    plsc.subcore_barrier()
    pltpu.sync_copy(acc_sh.at[pl.ds(sid * RPT, RPT)],
                    out_hbm.at[cid, pl.ds(sid * RPT, RPT)])


def _dis_from_degp(degp):
    deg = 1.0 + degp[0] + degp[1]
    return lax.rsqrt(deg)


def _tc1_body(x_ref, w1_ref, degp_ref, o_ref):
    dis = _dis_from_degp(degp_ref[...])
    h = jnp.dot(x_ref[...], w1_ref[...], preferred_element_type=jnp.float32)
    o_ref[...] = h * dis


_tc1 = pl.pallas_call(
    _tc1_body, out_shape=jax.ShapeDtypeStruct((N, H), jnp.float32))


def _bn_relu(z, g, be):
    mean = jnp.mean(z, axis=0, keepdims=True)
    c = z - mean
    var = jnp.mean(c * c, axis=0, keepdims=True)
    return jnp.maximum(c * lax.rsqrt(var + EPS) * g + be, 0.0)


def _tc2_body(part_ref, hp_ref, degp_ref, b_ref, g_ref, be_ref, w2_ref, o_ref):
    dis = _dis_from_degp(degp_ref[...])
    part = part_ref[...]
    z = (part[0, :N] + part[1, :N] + hp_ref[...]) * dis + b_ref[...]
    y = _bn_relu(z, g_ref[...], be_ref[...])
    o_ref[...] = jnp.dot(y, w2_ref[...], preferred_element_type=jnp.float32) * dis


_tc2 = pl.pallas_call(
    _tc2_body, out_shape=jax.ShapeDtypeStruct((N, H), jnp.float32))


def _tc3_body(part_ref, hp_ref, degp_ref, b_ref, g_ref, be_ref, wc_ref,
              bc_ref, o_ref):
    dis = _dis_from_degp(degp_ref[...])
    part = part_ref[...]
    z = (part[0, :N] + part[1, :N] + hp_ref[...]) * dis + b_ref[...]
    y = _bn_relu(z, g_ref[...], be_ref[...])
    logits = jnp.dot(y, wc_ref[...], preferred_element_type=jnp.float32) + bc_ref[...]
    m = jnp.max(logits, axis=1, keepdims=True)
    s = logits - m
    lse = jnp.log(jnp.sum(jnp.exp(s), axis=1, keepdims=True))
    o_ref[...] = s - lse


_tc3 = pl.pallas_call(
    _tc3_body, out_shape=jax.ShapeDtypeStruct((N, C), jnp.float32))


def kernel(x, edge_index, W1, b1, g1, be1, W2, b2, g2, be2, Wc, bc):
    ei = edge_index.astype(jnp.int32)
    src1 = ei[0]
    dst2 = ei[1].reshape(NW, NCHUNK, CH)
    dst2 = jnp.pad(dst2, ((0, 0), (0, NCP - NCHUNK), (0, 0)))
    zeros_nh = jnp.zeros((NP, H), jnp.float32)
    zeros_1d = jnp.zeros((NP,), jnp.float32)
    ones_c = jnp.ones((CH,), jnp.float32)

    degp = _deg_kernel(dst2, zeros_1d, ones_c)
    degp = degp.reshape(NC, NP, 1)[:, :N]
    h1p = _tc1(x, W1, degp)
    pA = _agg_kernel(h1p, src1, dst2, zeros_nh)
    h2p = _tc2(pA, h1p, degp, b1.reshape(1, H), g1.reshape(1, H),
               be1.reshape(1, H), W2)
    pB = _agg_kernel(h2p, src1, dst2, zeros_nh)
    out = _tc3(pB, h2p, degp, b2.reshape(1, H), g2.reshape(1, H),
               be2.reshape(1, H), Wc, bc.reshape(1, C))
    return out


# speedup vs baseline: 32.4150x; 1.0007x over previous
import functools

import jax
import jax.numpy as jnp
from jax import lax
from jax.experimental import pallas as pl
from jax.experimental.pallas import tpu as pltpu
from jax.experimental.pallas import tpu_sc as plsc

N = 10000
E = 320000
D = 128
H = 128
C = 40
EPS = 1e-5

NC = 2
NS = 16
NW = NC * NS
EPT = E // NW
CH = 80
NCHUNK = EPT // CH
NBUF = 3
NG = (NCHUNK + 7) // 8
NCP = NG * 8
NP = 10240
RPT = NP // NS

_mesh = plsc.VectorSubcoreMesh(
    core_axis_name="c", subcore_axis_name="s", num_cores=NC, num_subcores=NS)


@functools.partial(
    pl.kernel,
    out_type=jax.ShapeDtypeStruct((NC, 1, NP), jnp.float32),
    mesh=_mesh,
    scratch_types=[
        pltpu.VMEM((NCP, CH), jnp.int32),
        pltpu.VMEM((CH,), jnp.float32),
        pltpu.VMEM_SHARED((NP,), jnp.float32),
    ],
)
def _deg_kernel(dst3_hbm, zeros_hbm, ones_hbm, out_hbm, dsts_v, ones_v, acc_sh):
    cid = lax.axis_index("c")
    sid = lax.axis_index("s")
    wid = sid * NC + cid

    pltpu.sync_copy(ones_hbm, ones_v)
    pltpu.sync_copy(dst3_hbm.at[wid], dsts_v)
    pltpu.sync_copy(zeros_hbm.at[pl.ds(sid * RPT, RPT)],
                    acc_sh.at[pl.ds(sid * RPT, RPT)])
    plsc.subcore_barrier()

    @pl.loop(0, NCHUNK)
    def _chunks(j):
        pltpu.sync_copy(ones_v, acc_sh.at[dsts_v.at[j]], add=True)

    plsc.subcore_barrier()
    pltpu.sync_copy(acc_sh.at[pl.ds(sid * RPT, RPT)],
                    out_hbm.at[cid, 0, pl.ds(sid * RPT, RPT)])


@functools.partial(
    pl.kernel,
    out_type=jax.ShapeDtypeStruct((NC, NP, H), jnp.float32),
    mesh=_mesh,
    scratch_types=[
        pltpu.VMEM((EPT,), jnp.int32),
        pltpu.VMEM((2, 8, CH), jnp.int32),
        pltpu.VMEM((NBUF, CH, H), jnp.float32),
        pltpu.VMEM_SHARED((NP, H), jnp.float32),
        pltpu.SemaphoreType.DMA,
        pltpu.SemaphoreType.DMA,
        pltpu.SemaphoreType.DMA,
        pltpu.SemaphoreType.DMA,
        pltpu.SemaphoreType.DMA,
        pltpu.SemaphoreType.DMA,
        pltpu.SemaphoreType.DMA,
        pltpu.SemaphoreType.DMA,
    ],
)
def _agg_kernel(table_hbm, src1_hbm, dst3_hbm, zeros_hbm, out_hbm,
                srcs_v, dring_v, rows_v, acc_sh,
                g0, g1, g2, s0, s1, s2, d0, d1):
    gsem = [g0, g1, g2]
    ssem = [s0, s1, s2]
    dsem = [d0, d1]
    cid = lax.axis_index("c")
    sid = lax.axis_index("s")
    wid = sid * NC + cid

    pltpu.sync_copy(src1_hbm.at[pl.ds(wid * EPT, EPT)], srcs_v)
    pltpu.sync_copy(dst3_hbm.at[wid, pl.ds(0, 8)], dring_v.at[0])
    pltpu.async_copy(dst3_hbm.at[wid, pl.ds(8, 8)], dring_v.at[1], dsem[1])
    pltpu.sync_copy(zeros_hbm.at[pl.ds(0, CH)], rows_v.at[0])

    @pl.loop(0, RPT // CH)
    def _zinit(i):
        pltpu.sync_copy(rows_v.at[0],
                        acc_sh.at[pl.ds(sid * RPT + i * CH, CH)])

    plsc.subcore_barrier()

    for b in range(NBUF - 1):
        pltpu.async_copy(table_hbm.at[srcs_v.at[pl.ds(b * CH, CH)]],
                         rows_v.at[b], gsem[b])

    @pl.loop(0, NCHUNK, step=NBUF)
    def _outer(o):
        for b in range(NBUF):
            j = o + b

            @pl.when(j < NCHUNK)
            def _():
                g = lax.shift_right_logical(j, 3)
                gb = lax.bitwise_and(g, 1)
                r = lax.bitwise_and(j, 7)

                @pl.when((r == 0) & (g >= 1))
                def _():
                    @pl.when(gb == 0)
                    def _():
                        pltpu.make_async_copy(
                            dst3_hbm.at[wid, pl.ds(0, 8)], dring_v.at[0],
                            dsem[0]).wait()

                    @pl.when(gb == 1)
                    def _():
                        pltpu.make_async_copy(
                            dst3_hbm.at[wid, pl.ds(0, 8)], dring_v.at[1],
                            dsem[1]).wait()

                pltpu.make_async_copy(
                    table_hbm.at[srcs_v.at[pl.ds(j * CH, CH)]],
                    rows_v.at[b], gsem[b]).wait()
                pltpu.async_copy(rows_v.at[b], acc_sh.at[dring_v.at[gb, r]],
                                 ssem[b], add=True)

                k = j + NBUF - 1
                bk = (b + NBUF - 1) % NBUF

                @pl.when(k < NCHUNK)
                def _():
                    @pl.when(j >= 1)
                    def _():
                        pltpu.make_async_copy(
                            rows_v.at[bk], acc_sh.at[dring_v.at[0, 0]],
                            ssem[bk]).wait()

                    pltpu.async_copy(
                        table_hbm.at[srcs_v.at[pl.ds(k * CH, CH)]],
                        rows_v.at[bk], gsem[bk])

                @pl.when((r == 0) & (g >= 1) & (g < NG - 1))
                def _():
                    nxt = pl.multiple_of((g + 1) * 8, 8)

                    @pl.when(gb == 0)
                    def _():
                        pltpu.async_copy(dst3_hbm.at[wid, pl.ds(nxt, 8)],
                                         dring_v.at[1], dsem[1])

                    @pl.when(gb == 1)
                    def _():
                        pltpu.async_copy(dst3_hbm.at[wid, pl.ds(nxt, 8)],
                                         dring_v.at[0], dsem[0])

    for b in range(NBUF):
        pltpu.make_async_copy(rows_v.at[b], acc_sh.at[dring_v.at[0, 0]],
                              ssem[b]).wait()

    plsc.subcore_barrier()
    pltpu.sync_copy(acc_sh.at[pl.ds(sid * RPT, RPT)],
                    out_hbm.at[cid, pl.ds(sid * RPT, RPT)])


def _dis_from_degp(degp):
    deg = 1.0 + degp[0] + degp[1]
    return lax.rsqrt(deg)


def _tc1a_body(x_ref, w1_ref, o_ref):
    o_ref[...] = jnp.dot(x_ref[...], w1_ref[...],
                         preferred_element_type=jnp.float32)


_tc1a = pl.pallas_call(
    _tc1a_body, out_shape=jax.ShapeDtypeStruct((N, H), jnp.float32))


def _tc1b_body(h_ref, degp_ref, o_ref):
    o_ref[...] = h_ref[...] * _dis_from_degp(degp_ref[...])


_tc1b = pl.pallas_call(
    _tc1b_body, out_shape=jax.ShapeDtypeStruct((N, H), jnp.float32))


def _bn_relu(z, g, be):
    mean = jnp.mean(z, axis=0, keepdims=True)
    c = z - mean
    var = jnp.mean(c * c, axis=0, keepdims=True)
    return jnp.maximum(c * lax.rsqrt(var + EPS) * g + be, 0.0)


def _tc2_body(part_ref, hp_ref, degp_ref, b_ref, g_ref, be_ref, w2_ref, o_ref):
    dis = _dis_from_degp(degp_ref[...])
    part = part_ref[...]
    z = (part[0, :N] + part[1, :N] + hp_ref[...]) * dis + b_ref[...]
    y = _bn_relu(z, g_ref[...], be_ref[...])
    o_ref[...] = jnp.dot(y, w2_ref[...], preferred_element_type=jnp.float32) * dis


_tc2 = pl.pallas_call(
    _tc2_body, out_shape=jax.ShapeDtypeStruct((N, H), jnp.float32))


def _tc3_body(part_ref, hp_ref, degp_ref, b_ref, g_ref, be_ref, wc_ref,
              bc_ref, o_ref):
    dis = _dis_from_degp(degp_ref[...])
    part = part_ref[...]
    z = (part[0, :N] + part[1, :N] + hp_ref[...]) * dis + b_ref[...]
    y = _bn_relu(z, g_ref[...], be_ref[...])
    logits = jnp.dot(y, wc_ref[...], preferred_element_type=jnp.float32) + bc_ref[...]
    m = jnp.max(logits, axis=1, keepdims=True)
    s = logits - m
    lse = jnp.log(jnp.sum(jnp.exp(s), axis=1, keepdims=True))
    o_ref[...] = s - lse


_tc3 = pl.pallas_call(
    _tc3_body, out_shape=jax.ShapeDtypeStruct((N, C), jnp.float32))


def kernel(x, edge_index, W1, b1, g1, be1, W2, b2, g2, be2, Wc, bc):
    ei = edge_index.astype(jnp.int32)
    src1 = ei[0]
    dst2 = ei[1].reshape(NW, NCHUNK, CH)
    dst2 = jnp.pad(dst2, ((0, 0), (0, NCP - NCHUNK), (0, 0)))
    zeros_nh = jnp.zeros((NP, H), jnp.float32)
    zeros_1d = jnp.zeros((NP,), jnp.float32)
    ones_c = jnp.ones((CH,), jnp.float32)

    degp = _deg_kernel(dst2, zeros_1d, ones_c)
    h1 = _tc1a(x, W1)
    degp = degp.reshape(NC, NP, 1)[:, :N]
    h1p = _tc1b(h1, degp)
    pA = _agg_kernel(h1p, src1, dst2, zeros_nh)
    h2p = _tc2(pA, h1p, degp, b1.reshape(1, H), g1.reshape(1, H),
               be1.reshape(1, H), W2)
    pB = _agg_kernel(h2p, src1, dst2, zeros_nh)
    out = _tc3(pB, h2p, degp, b2.reshape(1, H), g2.reshape(1, H),
               be2.reshape(1, H), Wc, bc.reshape(1, C))
    return out
